# baseline (speedup 1.0000x reference)
import functools

import jax
import jax.numpy as jnp
from jax.experimental import pallas as pl
from jax.experimental.pallas import tpu as pltpu

_K = 8
_BB = 2


def _se_kernel(x_hbm, w1t_ref, w2_ref, o_hbm, ibuf, obuf, isem, osem,
               *, n, per, bb, k, inv_hw):
    b = pl.program_id(0)
    islot = jax.lax.rem(b, 2)

    def load_copy(j, step, sl):
        return pltpu.make_async_copy(
            x_hbm.at[pl.ds(j * per + step * bb, bb)],
            ibuf.at[sl, j],
            isem.at[sl, j],
        )

    def store_copy(j, step, sl):
        return pltpu.make_async_copy(
            obuf.at[sl, j],
            o_hbm.at[pl.ds(j * per + step * bb, bb)],
            osem.at[sl, j],
        )

    @pl.when(b == 0)
    def _prologue():
        for j in range(k):
            load_copy(j, 0, 0).start()

    @pl.when(b + 1 < n)
    def _prefetch():
        for j in range(k):
            load_copy(j, b + 1, 1 - islot).start()

    @pl.when(b >= 2)
    def _drain_prev():
        for j in range(k):
            store_copy(j, b - 2, islot).wait()

    w1t = w1t_ref[...].astype(jnp.float32)
    w2v = w2_ref[...].astype(jnp.float32)

    def _body(sl):
        for j in range(k):
            load_copy(j, b, sl).wait()
            x = ibuf[sl, j].astype(jnp.float32)
            avg = jnp.sum(x, axis=-1, keepdims=True) * inv_hw
            h = jnp.sum(w1t[None] * avg, axis=1, keepdims=True)
            h = jnp.maximum(h, 0.0)
            y = jnp.sum(w2v[None] * h, axis=-1, keepdims=True)
            obuf[sl, j] = x * jax.nn.sigmoid(y)
            store_copy(j, b, sl).start()

    @pl.when(islot == 0)
    def _even():
        _body(0)

    @pl.when(islot == 1)
    def _odd():
        _body(1)

    @pl.when(b == n - 1)
    def _drain_tail():
        for j in range(k):
            store_copy(j, b - 1, 1 - islot).wait()
        for j in range(k):
            store_copy(j, b, islot).wait()


def kernel(x, w1, w2):
    B, C, H, W = x.shape
    Cr = w1.shape[0]
    HW = H * W
    x_flat = x.reshape(B, C, HW)

    k = _K
    while B % (2 * k):
        k //= 2
    bb = _BB
    per = B // k
    n = per // bb

    w1t = jnp.transpose(w1)
    out = pl.pallas_call(
        functools.partial(
            _se_kernel, n=n, per=per, bb=bb, k=k, inv_hw=1.0 / float(HW)
        ),
        out_shape=jax.ShapeDtypeStruct((B, C, HW), x.dtype),
        grid=(n,),
        in_specs=[
            pl.BlockSpec(memory_space=pl.ANY),
            pl.BlockSpec((C, Cr), lambda b: (0, 0)),
            pl.BlockSpec((C, Cr), lambda b: (0, 0)),
        ],
        out_specs=pl.BlockSpec(memory_space=pl.ANY),
        scratch_shapes=[
            pltpu.VMEM((2, k, bb, C, HW), x.dtype),
            pltpu.VMEM((2, k, bb, C, HW), x.dtype),
            pltpu.SemaphoreType.DMA((2, k)),
            pltpu.SemaphoreType.DMA((2, k)),
        ],
        compiler_params=pltpu.CompilerParams(
            dimension_semantics=("arbitrary",),
            vmem_limit_bytes=56 << 20,
        ),
    )(x_flat, w1t, w2)
    return out.reshape(B, C, H, W)
